# baseline (speedup 1.0000x reference)
import functools

import jax
import jax.numpy as jnp
from jax import lax
from jax.experimental import pallas as pl
from jax.experimental.pallas import tpu as pltpu
from jax.experimental.pallas import tpu_sc as plsc

D_MODEL = 768
D_FF = 3072
E = 8
TOPK = 2
T = 2048

B = 256
NB = 24
NBB = NB * B

NC = 2
NS = 16
NW = NC * NS
ROWS_W = NBB // NW
TOK_W = T // NW


def _router_body(x_ref, rw_ref, rb_ref,
                 posa_ref, posb_ref, cwa_ref, cwb_ref,
                 be_ref, bv_ref, aux_ref):
    logits = lax.dot_general(rw_ref[...], x_ref[...],
                             ((( 0,), (1,)), ((), ())),
                             preferred_element_type=jnp.float32)
    logits = logits + rb_ref[...].reshape(E, 1)
    eidx = lax.broadcasted_iota(jnp.int32, (E, T), 0)
    m1 = jnp.max(logits, axis=0, keepdims=True)
    i1 = jnp.min(jnp.where(logits == m1, eidx, E), axis=0, keepdims=True)
    mask1 = eidx == i1
    l2 = jnp.where(mask1, float("-inf"), logits)
    m2 = jnp.max(l2, axis=0, keepdims=True)
    i2 = jnp.min(jnp.where(l2 == m2, eidx, E), axis=0, keepdims=True)
    mask2 = eidx == i2
    sel = mask1 | mask2
    z = jnp.exp(logits - m1)
    w = z / jnp.sum(z, axis=0, keepdims=True)
    wsel = jnp.where(sel, w, 0.0)
    c = wsel / jnp.sum(wsel, axis=0, keepdims=True)

    s = sel.astype(jnp.float32)
    k = 1
    while k < T:
        s = s + jnp.concatenate(
            [jnp.zeros((E, k), jnp.float32), s[:, :T - k]], axis=1)
        k *= 2
    cnt = s[:, T - 1:]
    rank = s - sel.astype(jnp.float32)

    cnti = cnt.astype(jnp.int32)
    nbi = (cnti + (B - 1)) // B
    off_blk = jnp.zeros((E, 1), jnp.int32)
    for j in range(1, E):
        off_blk = off_blk + jnp.concatenate(
            [jnp.zeros((j, 1), jnp.int32), nbi[:E - j]], axis=0)
    off_rows = (off_blk * B).astype(jnp.float32)

    pos = rank + off_rows
    posa = jnp.sum(jnp.where(mask1, pos, 0.0), axis=0, keepdims=True)
    posb = jnp.sum(jnp.where(mask2, pos, 0.0), axis=0, keepdims=True)
    posa_ref[...] = posa.astype(jnp.int32)
    posb_ref[...] = posb.astype(jnp.int32)
    cwa_ref[...] = jnp.sum(jnp.where(mask1, c, 0.0), axis=0, keepdims=True)
    cwb_ref[...] = jnp.sum(jnp.where(mask2, c, 0.0), axis=0, keepdims=True)

    cum_incl = off_blk + nbi
    total_nb = jnp.sum(nbi, axis=0, keepdims=True)
    biota = lax.broadcasted_iota(jnp.int32, (E, NB), 1)
    be = jnp.sum((biota >= cum_incl).astype(jnp.int32), axis=0, keepdims=True)
    be_ref[...] = jnp.minimum(be, E - 1)
    bv_ref[...] = (lax.broadcasted_iota(jnp.int32, (1, NB), 1)
                   < total_nb).astype(jnp.int32)

    util = cnt / jnp.float32(T * TOPK)
    mu = jnp.mean(util)
    var = jnp.sum((util - mu) ** 2) / jnp.float32(E - 1)
    cv = jnp.sqrt(var) / (mu + 1e-6)
    aux_ref[...] = jnp.broadcast_to(cv * cv, (1, 1))


def _router(xf, router_w, router_b, interpret=False):
    return pl.pallas_call(
        _router_body,
        out_shape=[
            jax.ShapeDtypeStruct((1, T), jnp.int32),
            jax.ShapeDtypeStruct((1, T), jnp.int32),
            jax.ShapeDtypeStruct((1, T), jnp.float32),
            jax.ShapeDtypeStruct((1, T), jnp.float32),
            jax.ShapeDtypeStruct((1, NB), jnp.int32),
            jax.ShapeDtypeStruct((1, NB), jnp.int32),
            jax.ShapeDtypeStruct((1, 1), jnp.float32),
        ],
        interpret=interpret,
    )(xf, router_w, router_b.reshape(1, E))


def _dispatch_body(posa_hbm, posb_hbm, cwa_hbm, cwb_hbm, xf_hbm,
                   xs_hbm, ws_hbm,
                   pa_v, pb_v, wa_v, wb_v, tok_v, wsrt_v,
                   idx0_v, idx1_v, rows_v, tok_sh, sem):
    cid = lax.axis_index("c")
    sid = lax.axis_index("s")

    @pl.when(sid == 0)
    def _scatter():
        pltpu.sync_copy(posa_hbm, pa_v)
        pltpu.sync_copy(posb_hbm, pb_v)
        pltpu.sync_copy(cwa_hbm, wa_v)
        pltpu.sync_copy(cwb_hbm, wb_v)

        def _init(i, _):
            tok_v[pl.ds(i * 16, 16)] = jnp.zeros((16,), jnp.int32)
            wsrt_v[pl.ds(i * 16, 16)] = jnp.zeros((16,), jnp.float32)
            return _
        lax.fori_loop(0, NBB // 16, _init, 0)

        def _scat(i, _):
            tid = jnp.int32(i * 16) + lax.iota(jnp.int32, 16)
            pa = pa_v[pl.ds(i * 16, 16)]
            pb = pb_v[pl.ds(i * 16, 16)]
            plsc.store_scatter(tok_v, [pa], tid)
            plsc.store_scatter(wsrt_v, [pa], wa_v[pl.ds(i * 16, 16)])
            plsc.store_scatter(tok_v, [pb], tid)
            plsc.store_scatter(wsrt_v, [pb], wb_v[pl.ds(i * 16, 16)])
            return _
        lax.fori_loop(0, T // 16, _scat, 0)

        pltpu.sync_copy(tok_v, tok_sh)

        @pl.when(cid == 0)
        def _():
            pltpu.sync_copy(wsrt_v, ws_hbm)

    plsc.subcore_barrier()

    wid = sid * NC + cid
    base = wid * ROWS_W
    half = ROWS_W // 2
    pltpu.sync_copy(tok_sh.at[pl.ds(base, half)], idx0_v)
    pltpu.sync_copy(tok_sh.at[pl.ds(base + half, half)], idx1_v)
    pltpu.async_copy(xf_hbm.at[idx0_v], rows_v, sem).wait()
    pltpu.sync_copy(rows_v, xs_hbm.at[pl.ds(base, half)])
    pltpu.async_copy(xf_hbm.at[idx1_v], rows_v, sem).wait()
    pltpu.sync_copy(rows_v, xs_hbm.at[pl.ds(base + half, half)])


def _dispatch(posa, posb, cwa, cwb, xf):
    mesh = plsc.VectorSubcoreMesh(core_axis_name="c", subcore_axis_name="s")
    f = pl.kernel(
        _dispatch_body,
        out_type=[
            jax.ShapeDtypeStruct((NBB, D_MODEL), jnp.float32),
            jax.ShapeDtypeStruct((NBB,), jnp.float32),
        ],
        mesh=mesh,
        scratch_types=[
            pltpu.VMEM((T,), jnp.int32),
            pltpu.VMEM((T,), jnp.int32),
            pltpu.VMEM((T,), jnp.float32),
            pltpu.VMEM((T,), jnp.float32),
            pltpu.VMEM((NBB,), jnp.int32),
            pltpu.VMEM((NBB,), jnp.float32),
            pltpu.VMEM((ROWS_W // 2,), jnp.int32),
            pltpu.VMEM((ROWS_W // 2,), jnp.int32),
            pltpu.VMEM((ROWS_W // 2, D_MODEL), jnp.float32),
            pltpu.MemorySpace.VMEM_SHARED((NBB,), jnp.int32),
            pltpu.SemaphoreType.DMA,
        ],
        compiler_params=pltpu.CompilerParams(needs_layout_passes=False),
    )
    return f(posa, posb, cwa, cwb, xf)


def _expert_body(be_ref, bv_ref, xs_ref, w1_ref, b1_ref, w2_ref, b2_ref,
                 ws_ref, ys_ref):
    b = pl.program_id(0)

    @pl.when(bv_ref[b] != 0)
    def _():
        xb = xs_ref[...]
        h = jnp.dot(xb, w1_ref[0], preferred_element_type=jnp.float32)
        h = h + b1_ref[0]
        h = 0.5 * h * (1.0 + lax.erf(h * 0.7071067811865476))
        y = jnp.dot(h, w2_ref[0], preferred_element_type=jnp.float32)
        y = y + b2_ref[0]
        ys_ref[...] = ws_ref[...] * y


def _experts(be, bv, xs, ws, w1, b1, w2, b2, interpret=False):
    grid_spec = pltpu.PrefetchScalarGridSpec(
        num_scalar_prefetch=2,
        grid=(NB,),
        in_specs=[
            pl.BlockSpec((B, D_MODEL), lambda b, be, bv: (b, 0)),
            pl.BlockSpec((1, D_MODEL, D_FF), lambda b, be, bv: (be[b], 0, 0)),
            pl.BlockSpec((1, 1, D_FF), lambda b, be, bv: (be[b], 0, 0)),
            pl.BlockSpec((1, D_FF, D_MODEL), lambda b, be, bv: (be[b], 0, 0)),
            pl.BlockSpec((1, 1, D_MODEL), lambda b, be, bv: (be[b], 0, 0)),
            pl.BlockSpec((B, 1), lambda b, be, bv: (b, 0)),
        ],
        out_specs=pl.BlockSpec((B, D_MODEL), lambda b, be, bv: (b, 0)),
    )
    return pl.pallas_call(
        _expert_body,
        grid_spec=grid_spec,
        out_shape=jax.ShapeDtypeStruct((NBB, D_MODEL), jnp.float32),
        compiler_params=pltpu.CompilerParams(
            dimension_semantics=("arbitrary",),
            vmem_limit_bytes=56 * 1024 * 1024,
        ),
        interpret=interpret,
    )(be, bv, xs, w1, b1.reshape(E, 1, D_FF), w2, b2.reshape(E, 1, D_MODEL),
      ws.reshape(NBB, 1))


def _combine_body(ys_hbm, posa_hbm, posb_hbm, out_hbm,
                  pa_v, pb_v, ra_v, rb_v, sema, semb):
    cid = lax.axis_index("c")
    sid = lax.axis_index("s")
    wid = sid * NC + cid
    base = wid * TOK_W
    pltpu.sync_copy(posa_hbm.at[pl.ds(base, TOK_W)], pa_v)
    pltpu.sync_copy(posb_hbm.at[pl.ds(base, TOK_W)], pb_v)
    cpa = pltpu.async_copy(ys_hbm.at[pa_v], ra_v, sema)
    cpb = pltpu.async_copy(ys_hbm.at[pb_v], rb_v, semb)
    cpa.wait()
    cpb.wait()

    def _add(i, _):
        t = i // (D_MODEL // 16)
        k = (i % (D_MODEL // 16)) * 16
        ra_v[t, pl.ds(k, 16)] = ra_v[t, pl.ds(k, 16)] + rb_v[t, pl.ds(k, 16)]
        return _
    lax.fori_loop(0, TOK_W * (D_MODEL // 16), _add, 0)
    pltpu.sync_copy(ra_v, out_hbm.at[pl.ds(base, TOK_W)])


def _combine(ys, posa, posb):
    mesh = plsc.VectorSubcoreMesh(core_axis_name="c", subcore_axis_name="s")
    f = pl.kernel(
        _combine_body,
        out_type=jax.ShapeDtypeStruct((T, D_MODEL), jnp.float32),
        mesh=mesh,
        scratch_types=[
            pltpu.VMEM((TOK_W,), jnp.int32),
            pltpu.VMEM((TOK_W,), jnp.int32),
            pltpu.VMEM((TOK_W, D_MODEL), jnp.float32),
            pltpu.VMEM((TOK_W, D_MODEL), jnp.float32),
            pltpu.SemaphoreType.DMA,
            pltpu.SemaphoreType.DMA,
        ],
        compiler_params=pltpu.CompilerParams(needs_layout_passes=False),
    )
    return f(ys, posa, posb)


def kernel(x, router_w, router_b, w1, b1, w2, b2):
    orig_shape = x.shape
    xf = x.reshape(-1, D_MODEL)
    posa, posb, cwa, cwb, be, bv, aux = _router(xf, router_w, router_b)
    posa = posa.reshape(T)
    posb = posb.reshape(T)
    xs, ws = _dispatch(posa, posb, cwa.reshape(T), cwb.reshape(T), xf)
    ys = _experts(be.reshape(NB), bv.reshape(NB), xs, ws, w1, b1, w2, b2)
    out = _combine(ys, posa, posb)
    return out.reshape(orig_shape), aux[0, 0]


# speedup vs baseline: 1.6653x; 1.6653x over previous
import functools

import jax
import jax.numpy as jnp
from jax import lax
from jax.experimental import pallas as pl
from jax.experimental.pallas import tpu as pltpu
from jax.experimental.pallas import tpu_sc as plsc

D_MODEL = 768
D_FF = 3072
E = 8
TOPK = 2
T = 2048

B = 256
NB = 24
NBB = NB * B

NC = 2
NS = 16
NW = NC * NS
ROWS_W = NBB // NW
TOK_W = T // NW


def _router_body(x_ref, rw_ref, rb_ref,
                 posa_ref, posb_ref, cwa_ref, cwb_ref,
                 be_ref, bv_ref, aux_ref):
    logits = lax.dot_general(rw_ref[...], x_ref[...],
                             ((( 0,), (1,)), ((), ())),
                             preferred_element_type=jnp.float32)
    logits = logits + rb_ref[...].reshape(E, 1)
    eidx = lax.broadcasted_iota(jnp.int32, (E, T), 0)
    m1 = jnp.max(logits, axis=0, keepdims=True)
    i1 = jnp.min(jnp.where(logits == m1, eidx, E), axis=0, keepdims=True)
    mask1 = eidx == i1
    l2 = jnp.where(mask1, float("-inf"), logits)
    m2 = jnp.max(l2, axis=0, keepdims=True)
    i2 = jnp.min(jnp.where(l2 == m2, eidx, E), axis=0, keepdims=True)
    mask2 = eidx == i2
    sel = mask1 | mask2
    z = jnp.exp(logits - m1)
    w = z / jnp.sum(z, axis=0, keepdims=True)
    wsel = jnp.where(sel, w, 0.0)
    c = wsel / jnp.sum(wsel, axis=0, keepdims=True)

    s = sel.astype(jnp.float32)
    k = 1
    while k < T:
        s = s + jnp.concatenate(
            [jnp.zeros((E, k), jnp.float32), s[:, :T - k]], axis=1)
        k *= 2
    cnt = s[:, T - 1:]
    rank = s - sel.astype(jnp.float32)

    cnti = cnt.astype(jnp.int32)
    nbi = (cnti + (B - 1)) // B
    off_blk = jnp.zeros((E, 1), jnp.int32)
    for j in range(1, E):
        off_blk = off_blk + jnp.concatenate(
            [jnp.zeros((j, 1), jnp.int32), nbi[:E - j]], axis=0)
    off_rows = (off_blk * B).astype(jnp.float32)

    pos = rank + off_rows
    posa = jnp.sum(jnp.where(mask1, pos, 0.0), axis=0, keepdims=True)
    posb = jnp.sum(jnp.where(mask2, pos, 0.0), axis=0, keepdims=True)
    posa_ref[...] = posa.astype(jnp.int32)
    posb_ref[...] = posb.astype(jnp.int32)
    cwa_ref[...] = jnp.sum(jnp.where(mask1, c, 0.0), axis=0, keepdims=True)
    cwb_ref[...] = jnp.sum(jnp.where(mask2, c, 0.0), axis=0, keepdims=True)

    cum_incl = off_blk + nbi
    total_nb = jnp.sum(nbi, axis=0, keepdims=True)
    biota = lax.broadcasted_iota(jnp.int32, (E, NB), 1)
    be = jnp.sum((biota >= cum_incl).astype(jnp.int32), axis=0, keepdims=True)
    be_ref[...] = jnp.minimum(be, E - 1)
    bv_ref[...] = (lax.broadcasted_iota(jnp.int32, (1, NB), 1)
                   < total_nb).astype(jnp.int32)

    util = cnt / jnp.float32(T * TOPK)
    mu = jnp.mean(util)
    var = jnp.sum((util - mu) ** 2) / jnp.float32(E - 1)
    cv = jnp.sqrt(var) / (mu + 1e-6)
    aux_ref[...] = jnp.broadcast_to(cv * cv, (1, 1))


def _router(xf, router_w, router_b, interpret=False):
    return pl.pallas_call(
        _router_body,
        out_shape=[
            jax.ShapeDtypeStruct((1, T), jnp.int32),
            jax.ShapeDtypeStruct((1, T), jnp.int32),
            jax.ShapeDtypeStruct((1, T), jnp.float32),
            jax.ShapeDtypeStruct((1, T), jnp.float32),
            jax.ShapeDtypeStruct((1, NB), jnp.int32),
            jax.ShapeDtypeStruct((1, NB), jnp.int32),
            jax.ShapeDtypeStruct((1, 1), jnp.float32),
        ],
        interpret=interpret,
    )(xf, router_w, router_b.reshape(1, E))


def _dispatch_body(posa_hbm, posb_hbm, cwa_hbm, cwb_hbm, xf_hbm,
                   xs_hbm, ws_hbm,
                   pa_v, pb_v, wa_v, wb_v, paf_v, pbf_v, wsrt_v,
                   rows_v, sema, semb):
    cid = lax.axis_index("c")
    sid = lax.axis_index("s")
    wid = sid * NC + cid
    tbase = wid * TOK_W

    pltpu.sync_copy(posa_hbm.at[pl.ds(tbase, TOK_W)], pa_v)
    pltpu.sync_copy(posb_hbm.at[pl.ds(tbase, TOK_W)], pb_v)
    pltpu.sync_copy(xf_hbm.at[pl.ds(tbase, TOK_W)], rows_v)
    cpa = pltpu.async_copy(rows_v, xs_hbm.at[pa_v], sema)
    cpb = pltpu.async_copy(rows_v, xs_hbm.at[pb_v], semb)

    @pl.when(wid == 0)
    def _wscatter():
        pltpu.sync_copy(posa_hbm, paf_v)
        pltpu.sync_copy(posb_hbm, pbf_v)
        pltpu.sync_copy(cwa_hbm, wa_v)
        pltpu.sync_copy(cwb_hbm, wb_v)

        def _scat(i, _):
            pa = paf_v[pl.ds(i * 16, 16)]
            pb = pbf_v[pl.ds(i * 16, 16)]
            plsc.store_scatter(wsrt_v, [pa], wa_v[pl.ds(i * 16, 16)])
            plsc.store_scatter(wsrt_v, [pb], wb_v[pl.ds(i * 16, 16)])
            return _
        lax.fori_loop(0, T // 16, _scat, 0)
        pltpu.sync_copy(wsrt_v, ws_hbm)

    cpa.wait()
    cpb.wait()


def _dispatch(posa, posb, cwa, cwb, xf):
    mesh = plsc.VectorSubcoreMesh(core_axis_name="c", subcore_axis_name="s")
    f = pl.kernel(
        _dispatch_body,
        out_type=[
            jax.ShapeDtypeStruct((NBB, D_MODEL), jnp.float32),
            jax.ShapeDtypeStruct((NBB,), jnp.float32),
        ],
        mesh=mesh,
        scratch_types=[
            pltpu.VMEM((TOK_W,), jnp.int32),
            pltpu.VMEM((TOK_W,), jnp.int32),
            pltpu.VMEM((T,), jnp.float32),
            pltpu.VMEM((T,), jnp.float32),
            pltpu.VMEM((T,), jnp.int32),
            pltpu.VMEM((T,), jnp.int32),
            pltpu.VMEM((NBB,), jnp.float32),
            pltpu.VMEM((TOK_W, D_MODEL), jnp.float32),
            pltpu.SemaphoreType.DMA,
            pltpu.SemaphoreType.DMA,
        ],
        compiler_params=pltpu.CompilerParams(needs_layout_passes=False),
    )
    return f(posa, posb, cwa, cwb, xf)


def _expert_body(be_ref, bv_ref, xs_ref, w1_ref, b1_ref, w2_ref, b2_ref,
                 ws_ref, ys_ref):
    b = pl.program_id(0)

    @pl.when(bv_ref[b] != 0)
    def _():
        xb = xs_ref[...]
        h = jnp.dot(xb, w1_ref[0], preferred_element_type=jnp.float32)
        h = h + b1_ref[0]
        h = 0.5 * h * (1.0 + lax.erf(h * 0.7071067811865476))
        y = jnp.dot(h, w2_ref[0], preferred_element_type=jnp.float32)
        y = y + b2_ref[0]
        ys_ref[...] = ws_ref[...] * y


def _experts(be, bv, xs, ws, w1, b1, w2, b2, interpret=False):
    grid_spec = pltpu.PrefetchScalarGridSpec(
        num_scalar_prefetch=2,
        grid=(NB,),
        in_specs=[
            pl.BlockSpec((B, D_MODEL), lambda b, be, bv: (b, 0)),
            pl.BlockSpec((1, D_MODEL, D_FF), lambda b, be, bv: (be[b], 0, 0)),
            pl.BlockSpec((1, 1, D_FF), lambda b, be, bv: (be[b], 0, 0)),
            pl.BlockSpec((1, D_FF, D_MODEL), lambda b, be, bv: (be[b], 0, 0)),
            pl.BlockSpec((1, 1, D_MODEL), lambda b, be, bv: (be[b], 0, 0)),
            pl.BlockSpec((B, 1), lambda b, be, bv: (b, 0)),
        ],
        out_specs=pl.BlockSpec((B, D_MODEL), lambda b, be, bv: (b, 0)),
    )
    return pl.pallas_call(
        _expert_body,
        grid_spec=grid_spec,
        out_shape=jax.ShapeDtypeStruct((NBB, D_MODEL), jnp.float32),
        compiler_params=pltpu.CompilerParams(
            dimension_semantics=("arbitrary",),
            vmem_limit_bytes=56 * 1024 * 1024,
        ),
        interpret=interpret,
    )(be, bv, xs, w1, b1.reshape(E, 1, D_FF), w2, b2.reshape(E, 1, D_MODEL),
      ws.reshape(NBB, 1))


def _combine_body(ys_hbm, posa_hbm, posb_hbm, out_hbm,
                  pa_v, pb_v, ra_v, rb_v, sema, semb):
    cid = lax.axis_index("c")
    sid = lax.axis_index("s")
    wid = sid * NC + cid
    base = wid * TOK_W
    pltpu.sync_copy(posa_hbm.at[pl.ds(base, TOK_W)], pa_v)
    pltpu.sync_copy(posb_hbm.at[pl.ds(base, TOK_W)], pb_v)
    cpa = pltpu.async_copy(ys_hbm.at[pa_v], ra_v, sema)
    cpb = pltpu.async_copy(ys_hbm.at[pb_v], rb_v, semb)
    cpa.wait()
    cpb.wait()

    def _add(i, _):
        t = i // (D_MODEL // 16)
        k = (i % (D_MODEL // 16)) * 16
        ra_v[t, pl.ds(k, 16)] = ra_v[t, pl.ds(k, 16)] + rb_v[t, pl.ds(k, 16)]
        return _
    lax.fori_loop(0, TOK_W * (D_MODEL // 16), _add, 0)
    pltpu.sync_copy(ra_v, out_hbm.at[pl.ds(base, TOK_W)])


def _combine(ys, posa, posb):
    mesh = plsc.VectorSubcoreMesh(core_axis_name="c", subcore_axis_name="s")
    f = pl.kernel(
        _combine_body,
        out_type=jax.ShapeDtypeStruct((T, D_MODEL), jnp.float32),
        mesh=mesh,
        scratch_types=[
            pltpu.VMEM((TOK_W,), jnp.int32),
            pltpu.VMEM((TOK_W,), jnp.int32),
            pltpu.VMEM((TOK_W, D_MODEL), jnp.float32),
            pltpu.VMEM((TOK_W, D_MODEL), jnp.float32),
            pltpu.SemaphoreType.DMA,
            pltpu.SemaphoreType.DMA,
        ],
        compiler_params=pltpu.CompilerParams(needs_layout_passes=False),
    )
    return f(ys, posa, posb)


def kernel(x, router_w, router_b, w1, b1, w2, b2):
    orig_shape = x.shape
    xf = x.reshape(-1, D_MODEL)
    posa, posb, cwa, cwb, be, bv, aux = _router(xf, router_w, router_b)
    posa = posa.reshape(T)
    posb = posb.reshape(T)
    xs, ws = _dispatch(posa, posb, cwa.reshape(T), cwb.reshape(T), xf)
    ys = _experts(be.reshape(NB), bv.reshape(NB), xs, ws, w1, b1, w2, b2)
    out = _combine(ys, posa, posb)
    return out.reshape(orig_shape), aux[0, 0]


# speedup vs baseline: 2.0054x; 1.2042x over previous
import jax
import jax.numpy as jnp
from jax import lax
from jax.experimental import pallas as pl
from jax.experimental.pallas import tpu as pltpu
from jax.experimental.pallas import tpu_sc as plsc

D_MODEL = 768
D_FF = 3072
E = 8
TOPK = 2
T = 2048

B = 256
NB = 24
NBB = NB * B

NC = 2
NS = 16
NW = NC * NS
ROWS_W = NBB // NW
TOK_W = T // NW


def _router_body(x_ref, rw_ref, rb_ref,
                 posa_ref, posb_ref, cwa_ref, cwb_ref,
                 be_ref, bv_ref, aux_ref):
    logits = lax.dot_general(rw_ref[...], x_ref[...],
                             ((( 0,), (1,)), ((), ())),
                             preferred_element_type=jnp.float32)
    logits = logits + rb_ref[...].reshape(E, 1)
    eidx = lax.broadcasted_iota(jnp.int32, (E, T), 0)
    m1 = jnp.max(logits, axis=0, keepdims=True)
    i1 = jnp.min(jnp.where(logits == m1, eidx, E), axis=0, keepdims=True)
    mask1 = eidx == i1
    l2 = jnp.where(mask1, float("-inf"), logits)
    m2 = jnp.max(l2, axis=0, keepdims=True)
    i2 = jnp.min(jnp.where(l2 == m2, eidx, E), axis=0, keepdims=True)
    mask2 = eidx == i2
    sel = mask1 | mask2
    z = jnp.exp(logits - m1)
    w = z / jnp.sum(z, axis=0, keepdims=True)
    wsel = jnp.where(sel, w, 0.0)
    c = wsel / jnp.sum(wsel, axis=0, keepdims=True)

    s = sel.astype(jnp.float32)
    k = 1
    while k < T:
        s = s + jnp.concatenate(
            [jnp.zeros((E, k), jnp.float32), s[:, :T - k]], axis=1)
        k *= 2
    cnt = s[:, T - 1:]
    rank = s - sel.astype(jnp.float32)

    cnti = cnt.astype(jnp.int32)
    nbi = (cnti + (B - 1)) // B
    off_blk = jnp.zeros((E, 1), jnp.int32)
    for j in range(1, E):
        off_blk = off_blk + jnp.concatenate(
            [jnp.zeros((j, 1), jnp.int32), nbi[:E - j]], axis=0)
    off_rows = (off_blk * B).astype(jnp.float32)

    pos = rank + off_rows
    posa = jnp.sum(jnp.where(mask1, pos, 0.0), axis=0, keepdims=True)
    posb = jnp.sum(jnp.where(mask2, pos, 0.0), axis=0, keepdims=True)
    posa_ref[...] = posa.astype(jnp.int32)
    posb_ref[...] = posb.astype(jnp.int32)
    cwa_ref[...] = jnp.sum(jnp.where(mask1, c, 0.0), axis=0, keepdims=True)
    cwb_ref[...] = jnp.sum(jnp.where(mask2, c, 0.0), axis=0, keepdims=True)

    cum_incl = off_blk + nbi
    total_nb = jnp.sum(nbi, axis=0, keepdims=True)
    biota = lax.broadcasted_iota(jnp.int32, (E, NB), 1)
    be = jnp.sum((biota >= cum_incl).astype(jnp.int32), axis=0, keepdims=True)
    be_ref[...] = jnp.minimum(be, E - 1)
    bv_ref[...] = (lax.broadcasted_iota(jnp.int32, (1, NB), 1)
                   < total_nb).astype(jnp.int32)

    util = cnt / jnp.float32(T * TOPK)
    mu = jnp.mean(util)
    var = jnp.sum((util - mu) ** 2) / jnp.float32(E - 1)
    cv = jnp.sqrt(var) / (mu + 1e-6)
    aux_ref[...] = jnp.broadcast_to(cv * cv, (1, 1))


def _router(xf, router_w, router_b, interpret=False):
    return pl.pallas_call(
        _router_body,
        out_shape=[
            jax.ShapeDtypeStruct((1, T), jnp.int32),
            jax.ShapeDtypeStruct((1, T), jnp.int32),
            jax.ShapeDtypeStruct((1, T), jnp.float32),
            jax.ShapeDtypeStruct((1, T), jnp.float32),
            jax.ShapeDtypeStruct((1, NB), jnp.int32),
            jax.ShapeDtypeStruct((1, NB), jnp.int32),
            jax.ShapeDtypeStruct((1, 1), jnp.float32),
        ],
        interpret=interpret,
    )(xf, router_w, router_b.reshape(1, E))


def _dispatch_body(posa_hbm, posb_hbm, cwa_hbm, cwb_hbm, xf_hbm,
                   xs_hbm, ws_hbm,
                   pa_v, pb_v, wa_v, wb_v, paf_v, pbf_v, wsrt_v,
                   rows_v, sema, semb):
    cid = lax.axis_index("c")
    sid = lax.axis_index("s")
    wid = sid * NC + cid
    tbase = wid * TOK_W

    pltpu.sync_copy(posa_hbm.at[pl.ds(tbase, TOK_W)], pa_v)
    pltpu.sync_copy(posb_hbm.at[pl.ds(tbase, TOK_W)], pb_v)
    pltpu.sync_copy(xf_hbm.at[pl.ds(tbase, TOK_W)], rows_v)
    cpa = pltpu.async_copy(rows_v, xs_hbm.at[pa_v], sema)
    cpb = pltpu.async_copy(rows_v, xs_hbm.at[pb_v], semb)

    @pl.when(wid == 0)
    def _wscatter():
        pltpu.sync_copy(posa_hbm, paf_v)
        pltpu.sync_copy(posb_hbm, pbf_v)
        pltpu.sync_copy(cwa_hbm, wa_v)
        pltpu.sync_copy(cwb_hbm, wb_v)

        def _scat(i, _):
            pa = paf_v[pl.ds(i * 16, 16)]
            pb = pbf_v[pl.ds(i * 16, 16)]
            plsc.store_scatter(wsrt_v, [pa], wa_v[pl.ds(i * 16, 16)])
            plsc.store_scatter(wsrt_v, [pb], wb_v[pl.ds(i * 16, 16)])
            return _
        lax.fori_loop(0, T // 16, _scat, 0)
        pltpu.sync_copy(wsrt_v, ws_hbm)

    cpa.wait()
    cpb.wait()


def _dispatch(posa, posb, cwa, cwb, xf):
    mesh = plsc.VectorSubcoreMesh(core_axis_name="c", subcore_axis_name="s")
    f = pl.kernel(
        _dispatch_body,
        out_type=[
            jax.ShapeDtypeStruct((NBB, D_MODEL), jnp.float32),
            jax.ShapeDtypeStruct((NBB,), jnp.float32),
        ],
        mesh=mesh,
        scratch_types=[
            pltpu.VMEM((TOK_W,), jnp.int32),
            pltpu.VMEM((TOK_W,), jnp.int32),
            pltpu.VMEM((T,), jnp.float32),
            pltpu.VMEM((T,), jnp.float32),
            pltpu.VMEM((T,), jnp.int32),
            pltpu.VMEM((T,), jnp.int32),
            pltpu.VMEM((NBB,), jnp.float32),
            pltpu.VMEM((TOK_W, D_MODEL), jnp.float32),
            pltpu.SemaphoreType.DMA,
            pltpu.SemaphoreType.DMA,
        ],
        compiler_params=pltpu.CompilerParams(needs_layout_passes=False),
    )
    return f(posa, posb, cwa, cwb, xf)


def _expert_body(be_ref, bv_ref, xs_ref, b1_ref, b2_ref, ws_ref,
                 w1_hbm, w2_hbm, ys_ref, w1b, w2b, h_ref, s1, s2):
    b = pl.program_id(0)
    my_e = be_ref[b]

    def _cnt(j, acc):
        return acc + jnp.where(be_ref[j] != be_ref[j - 1], 1, 0)
    run_ord = lax.fori_loop(1, b + 1, _cnt, 0)
    slot = lax.rem(run_ord, 2)
    is_start = jnp.logical_or(
        b == 0, be_ref[b] != be_ref[jnp.maximum(b - 1, 0)])

    def _nxt(j, acc):
        hit = jnp.logical_and(be_ref[j] != my_e, acc == NB)
        return jnp.where(hit, j, acc)
    nxt_b = lax.fori_loop(b + 1, NB, _nxt, NB)
    has_nxt = nxt_b < NB
    nxt_e = be_ref[jnp.minimum(nxt_b, NB - 1)]

    def _issue(e, sl):
        pltpu.make_async_copy(w1_hbm.at[e], w1b.at[sl], s1.at[sl]).start()
        pltpu.make_async_copy(w2_hbm.at[e], w2b.at[sl], s2.at[sl]).start()

    def _wait1(sl):
        pltpu.make_async_copy(w1_hbm.at[0], w1b.at[sl], s1.at[sl]).wait()

    def _wait2(sl):
        pltpu.make_async_copy(w2_hbm.at[0], w2b.at[sl], s2.at[sl]).wait()

    @pl.when(b == 0)
    def _ramp():
        _issue(my_e, 0)

        @pl.when(has_nxt)
        def _():
            _issue(nxt_e, 1)
        _wait1(0)

    @pl.when(jnp.logical_and(is_start, b > 0))
    def _swap():
        _wait1(slot)

        @pl.when(has_nxt)
        def _():
            _issue(nxt_e, 1 - slot)

    @pl.when(bv_ref[b] != 0)
    def _first_matmul():
        xb = xs_ref[...]
        h = jnp.dot(xb, w1b[slot], preferred_element_type=jnp.float32)
        h = h + b1_ref[0]
        h_ref[...] = 0.5 * h * (1.0 + lax.erf(h * 0.7071067811865476))

    @pl.when(is_start)
    def _():
        _wait2(slot)

    @pl.when(bv_ref[b] != 0)
    def _second_matmul():
        y = jnp.dot(h_ref[...], w2b[slot], preferred_element_type=jnp.float32)
        y = y + b2_ref[0]
        ys_ref[...] = ws_ref[...] * y


def _experts(be, bv, xs, ws, w1, b1, w2, b2, interpret=False):
    grid_spec = pltpu.PrefetchScalarGridSpec(
        num_scalar_prefetch=2,
        grid=(NB,),
        in_specs=[
            pl.BlockSpec((B, D_MODEL), lambda b, be, bv: (b, 0)),
            pl.BlockSpec((1, 1, D_FF), lambda b, be, bv: (be[b], 0, 0)),
            pl.BlockSpec((1, 1, D_MODEL), lambda b, be, bv: (be[b], 0, 0)),
            pl.BlockSpec((B, 1), lambda b, be, bv: (b, 0)),
            pl.BlockSpec(memory_space=pltpu.MemorySpace.HBM),
            pl.BlockSpec(memory_space=pltpu.MemorySpace.HBM),
        ],
        out_specs=pl.BlockSpec((B, D_MODEL), lambda b, be, bv: (b, 0)),
        scratch_shapes=[
            pltpu.VMEM((2, D_MODEL, D_FF), jnp.float32),
            pltpu.VMEM((2, D_FF, D_MODEL), jnp.float32),
            pltpu.VMEM((B, D_FF), jnp.float32),
            pltpu.SemaphoreType.DMA((2,)),
            pltpu.SemaphoreType.DMA((2,)),
        ],
    )
    return pl.pallas_call(
        _expert_body,
        grid_spec=grid_spec,
        out_shape=jax.ShapeDtypeStruct((NBB, D_MODEL), jnp.float32),
        compiler_params=pltpu.CompilerParams(
            dimension_semantics=("arbitrary",),
            vmem_limit_bytes=56 * 1024 * 1024,
        ),
        interpret=interpret,
    )(be, bv, xs, b1.reshape(E, 1, D_FF), b2.reshape(E, 1, D_MODEL),
      ws.reshape(NBB, 1), w1, w2)


def _combine_body(ys_hbm, posa_hbm, posb_hbm, out_hbm,
                  pa_v, pb_v, ra_v, rb_v, sema, semb):
    cid = lax.axis_index("c")
    sid = lax.axis_index("s")
    wid = sid * NC + cid
    base = wid * TOK_W
    pltpu.sync_copy(posa_hbm.at[pl.ds(base, TOK_W)], pa_v)
    pltpu.sync_copy(posb_hbm.at[pl.ds(base, TOK_W)], pb_v)
    cpa = pltpu.async_copy(ys_hbm.at[pa_v], ra_v, sema)
    cpb = pltpu.async_copy(ys_hbm.at[pb_v], rb_v, semb)
    cpa.wait()
    cpb.wait()

    def _add(t, _):
        for kk in range(D_MODEL // 16):
            k = kk * 16
            ra_v[t, pl.ds(k, 16)] = (ra_v[t, pl.ds(k, 16)]
                                     + rb_v[t, pl.ds(k, 16)])
        return _
    lax.fori_loop(0, TOK_W, _add, 0)
    pltpu.sync_copy(ra_v, out_hbm.at[pl.ds(base, TOK_W)])


def _combine(ys, posa, posb):
    mesh = plsc.VectorSubcoreMesh(core_axis_name="c", subcore_axis_name="s")
    f = pl.kernel(
        _combine_body,
        out_type=jax.ShapeDtypeStruct((T, D_MODEL), jnp.float32),
        mesh=mesh,
        scratch_types=[
            pltpu.VMEM((TOK_W,), jnp.int32),
            pltpu.VMEM((TOK_W,), jnp.int32),
            pltpu.VMEM((TOK_W, D_MODEL), jnp.float32),
            pltpu.VMEM((TOK_W, D_MODEL), jnp.float32),
            pltpu.SemaphoreType.DMA,
            pltpu.SemaphoreType.DMA,
        ],
        compiler_params=pltpu.CompilerParams(needs_layout_passes=False),
    )
    return f(ys, posa, posb)


def kernel(x, router_w, router_b, w1, b1, w2, b2):
    orig_shape = x.shape
    xf = x.reshape(-1, D_MODEL)
    posa, posb, cwa, cwb, be, bv, aux = _router(xf, router_w, router_b)
    posa = posa.reshape(T)
    posb = posb.reshape(T)
    xs, ws = _dispatch(posa, posb, cwa.reshape(T), cwb.reshape(T), xf)
    ys = _experts(be.reshape(NB), bv.reshape(NB), xs, ws, w1, b1, w2, b2)
    out = _combine(ys, posa, posb)
    return out.reshape(orig_shape), aux[0, 0]


# speedup vs baseline: 2.0252x; 1.0098x over previous
import jax
import jax.numpy as jnp
from jax import lax
from jax.experimental import pallas as pl
from jax.experimental.pallas import tpu as pltpu
from jax.experimental.pallas import tpu_sc as plsc

D_MODEL = 768
D_FF = 3072
E = 8
TOPK = 2
T = 2048

B = 256
NB = 24
NBB = NB * B

NC = 2
NS = 16
NW = NC * NS
ROWS_W = NBB // NW
TOK_W = T // NW


def _router_body(x_ref, rw_ref, rb_ref,
                 posa_ref, posb_ref, cwa_ref, cwb_ref,
                 be_ref, bv_ref, aux_ref):
    logits = lax.dot_general(rw_ref[...], x_ref[...],
                             ((( 0,), (1,)), ((), ())),
                             preferred_element_type=jnp.float32)
    logits = logits + rb_ref[...].reshape(E, 1)
    eidx = lax.broadcasted_iota(jnp.int32, (E, T), 0)
    m1 = jnp.max(logits, axis=0, keepdims=True)
    i1 = jnp.min(jnp.where(logits == m1, eidx, E), axis=0, keepdims=True)
    mask1 = eidx == i1
    l2 = jnp.where(mask1, float("-inf"), logits)
    m2 = jnp.max(l2, axis=0, keepdims=True)
    i2 = jnp.min(jnp.where(l2 == m2, eidx, E), axis=0, keepdims=True)
    mask2 = eidx == i2
    sel = mask1 | mask2
    z = jnp.exp(logits - m1)
    w = z / jnp.sum(z, axis=0, keepdims=True)
    wsel = jnp.where(sel, w, 0.0)
    c = wsel / jnp.sum(wsel, axis=0, keepdims=True)

    s = sel.astype(jnp.float32)
    k = 1
    while k < T:
        s = s + jnp.concatenate(
            [jnp.zeros((E, k), jnp.float32), s[:, :T - k]], axis=1)
        k *= 2
    cnt = s[:, T - 1:]
    rank = s - sel.astype(jnp.float32)

    cnti = cnt.astype(jnp.int32)
    nbi = (cnti + (B - 1)) // B
    off_blk = jnp.zeros((E, 1), jnp.int32)
    for j in range(1, E):
        off_blk = off_blk + jnp.concatenate(
            [jnp.zeros((j, 1), jnp.int32), nbi[:E - j]], axis=0)
    off_rows = (off_blk * B).astype(jnp.float32)

    pos = rank + off_rows
    posa = jnp.sum(jnp.where(mask1, pos, 0.0), axis=0, keepdims=True)
    posb = jnp.sum(jnp.where(mask2, pos, 0.0), axis=0, keepdims=True)
    posa_ref[...] = posa.astype(jnp.int32)
    posb_ref[...] = posb.astype(jnp.int32)
    cwa_ref[...] = jnp.sum(jnp.where(mask1, c, 0.0), axis=0, keepdims=True)
    cwb_ref[...] = jnp.sum(jnp.where(mask2, c, 0.0), axis=0, keepdims=True)

    cum_incl = off_blk + nbi
    total_nb = jnp.sum(nbi, axis=0, keepdims=True)
    biota = lax.broadcasted_iota(jnp.int32, (E, NB), 1)
    be = jnp.sum((biota >= cum_incl).astype(jnp.int32), axis=0, keepdims=True)
    be_ref[...] = jnp.minimum(be, E - 1)
    bv_ref[...] = (lax.broadcasted_iota(jnp.int32, (1, NB), 1)
                   < total_nb).astype(jnp.int32)

    util = cnt / jnp.float32(T * TOPK)
    mu = jnp.mean(util)
    var = jnp.sum((util - mu) ** 2) / jnp.float32(E - 1)
    cv = jnp.sqrt(var) / (mu + 1e-6)
    aux_ref[...] = jnp.broadcast_to(cv * cv, (1, 1))


def _router(xf, router_w, router_b, interpret=False):
    return pl.pallas_call(
        _router_body,
        out_shape=[
            jax.ShapeDtypeStruct((1, T), jnp.int32),
            jax.ShapeDtypeStruct((1, T), jnp.int32),
            jax.ShapeDtypeStruct((1, T), jnp.float32),
            jax.ShapeDtypeStruct((1, T), jnp.float32),
            jax.ShapeDtypeStruct((1, NB), jnp.int32),
            jax.ShapeDtypeStruct((1, NB), jnp.int32),
            jax.ShapeDtypeStruct((1, 1), jnp.float32),
        ],
        interpret=interpret,
    )(xf, router_w, router_b.reshape(1, E))


def _dispatch_body(posa_hbm, posb_hbm, cwa_hbm, cwb_hbm, xf_hbm,
                   xs_hbm, ws_hbm,
                   pa_v, pb_v, wa_v, wb_v, paf_v, pbf_v, wsrt_v,
                   rows_v, sema, semb):
    cid = lax.axis_index("c")
    sid = lax.axis_index("s")
    wid = sid * NC + cid
    tbase = wid * TOK_W

    pltpu.sync_copy(posa_hbm.at[pl.ds(tbase, TOK_W)], pa_v)
    pltpu.sync_copy(posb_hbm.at[pl.ds(tbase, TOK_W)], pb_v)
    pltpu.sync_copy(xf_hbm.at[pl.ds(tbase, TOK_W)], rows_v)
    cpa = pltpu.async_copy(rows_v, xs_hbm.at[pa_v], sema)
    cpb = pltpu.async_copy(rows_v, xs_hbm.at[pb_v], semb)

    @pl.when(wid == 0)
    def _wscatter():
        pltpu.sync_copy(posa_hbm, paf_v)
        pltpu.sync_copy(posb_hbm, pbf_v)
        pltpu.sync_copy(cwa_hbm, wa_v)
        pltpu.sync_copy(cwb_hbm, wb_v)

        def _scat(i, _):
            pa = paf_v[pl.ds(i * 16, 16)]
            pb = pbf_v[pl.ds(i * 16, 16)]
            plsc.store_scatter(wsrt_v, [pa], wa_v[pl.ds(i * 16, 16)])
            plsc.store_scatter(wsrt_v, [pb], wb_v[pl.ds(i * 16, 16)])
            return _
        lax.fori_loop(0, T // 16, _scat, 0)
        pltpu.sync_copy(wsrt_v, ws_hbm)

    cpa.wait()
    cpb.wait()


def _dispatch(posa, posb, cwa, cwb, xf):
    mesh = plsc.VectorSubcoreMesh(core_axis_name="c", subcore_axis_name="s")
    f = pl.kernel(
        _dispatch_body,
        out_type=[
            jax.ShapeDtypeStruct((NBB, D_MODEL), jnp.float32),
            jax.ShapeDtypeStruct((NBB,), jnp.float32),
        ],
        mesh=mesh,
        scratch_types=[
            pltpu.VMEM((TOK_W,), jnp.int32),
            pltpu.VMEM((TOK_W,), jnp.int32),
            pltpu.VMEM((T,), jnp.float32),
            pltpu.VMEM((T,), jnp.float32),
            pltpu.VMEM((T,), jnp.int32),
            pltpu.VMEM((T,), jnp.int32),
            pltpu.VMEM((NBB,), jnp.float32),
            pltpu.VMEM((TOK_W, D_MODEL), jnp.float32),
            pltpu.SemaphoreType.DMA,
            pltpu.SemaphoreType.DMA,
        ],
        compiler_params=pltpu.CompilerParams(needs_layout_passes=False),
    )
    return f(posa, posb, cwa, cwb, xf)


def _expert_body(be_ref, bv_ref, xs_ref, b1_ref, b2_ref, ws_ref,
                 w1_hbm, w2_hbm, ys_ref, w1b, w2b, s1, s2):
    b = pl.program_id(0)
    my_e = be_ref[b]

    def _cnt(j, acc):
        return acc + jnp.where(be_ref[j] != be_ref[j - 1], 1, 0)
    run_ord = lax.fori_loop(1, b + 1, _cnt, 0)
    slot = lax.rem(run_ord, 2)
    is_start = jnp.logical_or(
        b == 0, be_ref[b] != be_ref[jnp.maximum(b - 1, 0)])

    def _nxt(j, acc):
        hit = jnp.logical_and(be_ref[j] != my_e, acc == NB)
        return jnp.where(hit, j, acc)
    nxt_b = lax.fori_loop(b + 1, NB, _nxt, NB)
    has_nxt = nxt_b < NB
    nxt_e = be_ref[jnp.minimum(nxt_b, NB - 1)]

    def _issue(e, sl):
        pltpu.make_async_copy(w1_hbm.at[e], w1b.at[sl], s1.at[sl]).start()
        pltpu.make_async_copy(w2_hbm.at[e], w2b.at[sl], s2.at[sl]).start()

    def _wait(sl):
        pltpu.make_async_copy(w1_hbm.at[0], w1b.at[sl], s1.at[sl]).wait()
        pltpu.make_async_copy(w2_hbm.at[0], w2b.at[sl], s2.at[sl]).wait()

    @pl.when(b == 0)
    def _ramp():
        _issue(my_e, 0)

        @pl.when(has_nxt)
        def _():
            _issue(nxt_e, 1)
        _wait(0)

    @pl.when(jnp.logical_and(is_start, b > 0))
    def _swap():
        _wait(slot)

        @pl.when(has_nxt)
        def _():
            _issue(nxt_e, 1 - slot)

    @pl.when(bv_ref[b] != 0)
    def _compute():
        xb = xs_ref[...]
        h = jnp.dot(xb, w1b[slot], preferred_element_type=jnp.float32)
        h = h + b1_ref[0]
        h = 0.5 * h * (1.0 + lax.erf(h * 0.7071067811865476))
        y = jnp.dot(h, w2b[slot], preferred_element_type=jnp.float32)
        y = y + b2_ref[0]
        ys_ref[...] = ws_ref[...] * y


def _experts(be, bv, xs, ws, w1, b1, w2, b2, interpret=False):
    grid_spec = pltpu.PrefetchScalarGridSpec(
        num_scalar_prefetch=2,
        grid=(NB,),
        in_specs=[
            pl.BlockSpec((B, D_MODEL), lambda b, be, bv: (b, 0)),
            pl.BlockSpec((1, 1, D_FF), lambda b, be, bv: (be[b], 0, 0)),
            pl.BlockSpec((1, 1, D_MODEL), lambda b, be, bv: (be[b], 0, 0)),
            pl.BlockSpec((B, 1), lambda b, be, bv: (b, 0)),
            pl.BlockSpec(memory_space=pltpu.MemorySpace.HBM),
            pl.BlockSpec(memory_space=pltpu.MemorySpace.HBM),
        ],
        out_specs=pl.BlockSpec((B, D_MODEL), lambda b, be, bv: (b, 0)),
        scratch_shapes=[
            pltpu.VMEM((2, D_MODEL, D_FF), jnp.float32),
            pltpu.VMEM((2, D_FF, D_MODEL), jnp.float32),
            pltpu.SemaphoreType.DMA((2,)),
            pltpu.SemaphoreType.DMA((2,)),
        ],
    )
    return pl.pallas_call(
        _expert_body,
        grid_spec=grid_spec,
        out_shape=jax.ShapeDtypeStruct((NBB, D_MODEL), jnp.float32),
        compiler_params=pltpu.CompilerParams(
            dimension_semantics=("arbitrary",),
            vmem_limit_bytes=56 * 1024 * 1024,
        ),
        interpret=interpret,
    )(be, bv, xs, b1.reshape(E, 1, D_FF), b2.reshape(E, 1, D_MODEL),
      ws.reshape(NBB, 1), w1, w2)


def _combine_body(ys_hbm, posa_hbm, posb_hbm, out_hbm,
                  pa_v, pb_v, ra_v, rb_v, sema, semb):
    cid = lax.axis_index("c")
    sid = lax.axis_index("s")
    wid = sid * NC + cid
    base = wid * TOK_W
    pltpu.sync_copy(posa_hbm.at[pl.ds(base, TOK_W)], pa_v)
    pltpu.sync_copy(posb_hbm.at[pl.ds(base, TOK_W)], pb_v)
    cpa = pltpu.async_copy(ys_hbm.at[pa_v], ra_v, sema)
    cpb = pltpu.async_copy(ys_hbm.at[pb_v], rb_v, semb)
    cpa.wait()
    cpb.wait()

    def _add(t, _):
        for kk in range(D_MODEL // 16):
            k = kk * 16
            ra_v[t, pl.ds(k, 16)] = (ra_v[t, pl.ds(k, 16)]
                                     + rb_v[t, pl.ds(k, 16)])
        return _
    lax.fori_loop(0, TOK_W, _add, 0)
    pltpu.sync_copy(ra_v, out_hbm.at[pl.ds(base, TOK_W)])


def _combine(ys, posa, posb):
    mesh = plsc.VectorSubcoreMesh(core_axis_name="c", subcore_axis_name="s")
    f = pl.kernel(
        _combine_body,
        out_type=jax.ShapeDtypeStruct((T, D_MODEL), jnp.float32),
        mesh=mesh,
        scratch_types=[
            pltpu.VMEM((TOK_W,), jnp.int32),
            pltpu.VMEM((TOK_W,), jnp.int32),
            pltpu.VMEM((TOK_W, D_MODEL), jnp.float32),
            pltpu.VMEM((TOK_W, D_MODEL), jnp.float32),
            pltpu.SemaphoreType.DMA,
            pltpu.SemaphoreType.DMA,
        ],
        compiler_params=pltpu.CompilerParams(needs_layout_passes=False),
    )
    return f(ys, posa, posb)


def kernel(x, router_w, router_b, w1, b1, w2, b2):
    orig_shape = x.shape
    xf = x.reshape(-1, D_MODEL)
    posa, posb, cwa, cwb, be, bv, aux = _router(xf, router_w, router_b)
    posa = posa.reshape(T)
    posb = posb.reshape(T)
    xs, ws = _dispatch(posa, posb, cwa.reshape(T), cwb.reshape(T), xf)
    ys = _experts(be.reshape(NB), bv.reshape(NB), xs, ws, w1, b1, w2, b2)
    out = _combine(ys, posa, posb)
    return out.reshape(orig_shape), aux[0, 0]
